# Initial kernel scaffold; baseline (speedup 1.0000x reference)
import jax
import jax.numpy as jnp
from jax.experimental import pallas as pl


def kernel(token_ids, position_table, charge_table, mass_table, valence_table, identity_table, seq_pos_table):
    raise NotImplementedError("write your pallas kernel here")


# speedup vs baseline: 13.9265x; 13.9265x over previous
import functools

import jax
import jax.numpy as jnp
from jax import lax
from jax.experimental import pallas as pl
from jax.experimental.pallas import tpu as pltpu
from jax.experimental.pallas import tpu_sc as plsc


def _softplus(x):
    return jnp.where(x > 20.0, x, jnp.log(1.0 + jnp.exp(jnp.minimum(x, 20.0))))


def _prep_tables(charge_p, mass_p, valence_p, seq_pos_table):
    def body(c_ref, m_ref, v_ref, s_ref, co_ref, mo_ref, vo_ref, so_ref):
        co_ref[...] = jnp.tanh(c_ref[...])
        mo_ref[...] = _softplus(m_ref[...]) + 0.5
        vo_ref[...] = _softplus(v_ref[...]) + 1.0
        so_ref[...] = s_ref[...] * 0.1

    return pl.pallas_call(
        body,
        out_shape=(
            jax.ShapeDtypeStruct(charge_p.shape, jnp.float32),
            jax.ShapeDtypeStruct(mass_p.shape, jnp.float32),
            jax.ShapeDtypeStruct(valence_p.shape, jnp.float32),
            jax.ShapeDtypeStruct(seq_pos_table.shape, jnp.float32),
        ),
    )(charge_p, mass_p, valence_p, seq_pos_table)


def _sc_lookup(ids_r, position_table, identity_table, c_act, m_act, v_act,
               seq_scaled, n_seq, n_cores, n_subcores):
    NW, nblk, T = ids_r.shape
    total = NW * nblk * T
    chunk = nblk * T
    PD = position_table.shape[1]
    ID = identity_table.shape[1]

    mesh = plsc.VectorSubcoreMesh(core_axis_name="c", subcore_axis_name="s",
                                  num_cores=n_cores, num_subcores=n_subcores)

    @functools.partial(
        pl.kernel,
        compiler_params=pltpu.CompilerParams(use_tc_tiling_on_sc=False),
        out_type=[
            jax.ShapeDtypeStruct((total, PD), jnp.float32),
            jax.ShapeDtypeStruct((total, ID), jnp.float32),
            jax.ShapeDtypeStruct((total,), jnp.float32),
            jax.ShapeDtypeStruct((total,), jnp.float32),
            jax.ShapeDtypeStruct((total,), jnp.float32),
        ],
        mesh=mesh,
        scratch_types=[
            pltpu.VMEM((nblk, T), jnp.int32),
            pltpu.VMEM((n_seq + T, PD), jnp.float32),
            pltpu.VMEM((T, PD), jnp.float32),
            pltpu.VMEM((T, ID), jnp.float32),
            pltpu.VMEM((chunk,), jnp.float32),
            pltpu.VMEM((chunk,), jnp.float32),
            pltpu.VMEM((chunk,), jnp.float32),
            pltpu.SemaphoreType.DMA,
            pltpu.SemaphoreType.DMA,
            pltpu.SemaphoreType.DMA,
            pltpu.SemaphoreType.DMA,
            pltpu.SemaphoreType.DMA,
        ],
    )
    def sc_kernel(ids_hbm, pos_hbm, id_hbm, c_hbm, m_hbm, v_hbm, seq_hbm,
                  pos_out, id_out, c_out, m_out, v_out,
                  idsv, seqext, posbuf, idbuf, cfull, mfull, vfull,
                  sem_p, sem_i, sem_c, sem_m, sem_v):
        wid = lax.axis_index("s") * n_cores + lax.axis_index("c")
        wbase = wid * chunk
        pltpu.sync_copy(ids_hbm.at[wid], idsv)
        pltpu.sync_copy(seq_hbm.at[pl.ds(0, n_seq)], seqext.at[pl.ds(0, n_seq)])
        pltpu.sync_copy(seq_hbm.at[pl.ds(0, T)], seqext.at[pl.ds(n_seq, T)])

        def blk(b, carry):
            idx = idsv.at[b]
            cp_p = pltpu.async_copy(pos_hbm.at[idx], posbuf, sem_p)
            cp_i = pltpu.async_copy(id_hbm.at[idx], idbuf, sem_i)
            cp_c = pltpu.async_copy(c_hbm.at[idx], cfull.at[pl.ds(b * T, T)], sem_c)
            cp_m = pltpu.async_copy(m_hbm.at[idx], mfull.at[pl.ds(b * T, T)], sem_m)
            cp_v = pltpu.async_copy(v_hbm.at[idx], vfull.at[pl.ds(b * T, T)], sem_v)
            cp_p.wait()
            base = lax.rem(b * T, n_seq)

            def row(i, c2):
                for j in range(PD // 16):
                    sl = pl.ds(j * 16, 16)
                    posbuf[i, sl] = posbuf[i, sl] + seqext[base + i, sl]
                return c2

            lax.fori_loop(0, T, row, 0)
            cp_i.wait()
            cp_c.wait()
            cp_m.wait()
            cp_v.wait()
            ob = wbase + b * T
            pltpu.sync_copy(posbuf, pos_out.at[pl.ds(ob, T)])
            pltpu.sync_copy(idbuf, id_out.at[pl.ds(ob, T)])
            return carry

        lax.fori_loop(0, nblk, blk, 0)
        pltpu.sync_copy(cfull, c_out.at[pl.ds(wbase, chunk)])
        pltpu.sync_copy(mfull, m_out.at[pl.ds(wbase, chunk)])
        pltpu.sync_copy(vfull, v_out.at[pl.ds(wbase, chunk)])

    return sc_kernel(ids_r, position_table, identity_table, c_act, m_act,
                     v_act, seq_scaled)


def kernel(token_ids, position_table, charge_table, mass_table, valence_table,
           identity_table, seq_pos_table):
    B, N = token_ids.shape
    V, PD = position_table.shape
    ID = identity_table.shape[1]
    total = B * N

    try:
        info = plsc.get_sparse_core_info()
        n_cores, n_subcores = info.num_cores, info.num_subcores
    except Exception:
        n_cores, n_subcores = 2, 16
    NW = n_cores * n_subcores
    T = 128
    chunk = total // NW
    nblk = chunk // T
    assert chunk * NW == total and nblk * T == chunk

    pad = (-V) % 128

    def as_rows(t):
        return jnp.pad(t.reshape(-1), (0, pad)).reshape(-1, 128)

    c_act, m_act, v_act, seq_scaled = _prep_tables(
        as_rows(charge_table), as_rows(mass_table), as_rows(valence_table),
        seq_pos_table)
    c_act = c_act.reshape(-1)
    m_act = m_act.reshape(-1)
    v_act = v_act.reshape(-1)

    ids_r = token_ids.reshape(NW, nblk, T)

    pos_out, id_out, c_out, m_out, v_out = _sc_lookup(
        ids_r, position_table, identity_table, c_act, m_act, v_act,
        seq_scaled, N, n_cores, n_subcores)

    charge = c_out.reshape(B, N, 1)
    position = pos_out.reshape(B, N, PD)
    momentum = jnp.zeros((B, N, PD), jnp.float32)
    mass = m_out.reshape(B, N, 1)
    valence = v_out.reshape(B, N, 1)
    identity = id_out.reshape(B, N, ID)
    return (charge, position, momentum, mass, valence, identity)
